# Initial kernel scaffold; baseline (speedup 1.0000x reference)
import jax
import jax.numpy as jnp
from jax.experimental import pallas as pl


def kernel(pts, W1, b1, W2, b2, W3, b3, W4, b4):
    raise NotImplementedError("write your pallas kernel here")


# speedup vs baseline: 6.1958x; 6.1958x over previous
import jax
import jax.numpy as jnp
from jax.experimental import pallas as pl

_K = 16
_OC = 128
_M = 256


def _mm(a, b):
    return jax.lax.dot_general(a, b, (((1,), (0,)), ((), ())),
                               preferred_element_type=jnp.float32)


def _nbr_kernel(pts_ref, ptst_ref, w1_ref, b1_ref, w2_ref, b2_ref, w3_ref,
                b3_ref, w4a_ref, w4b_ref, b4_ref, out_ref):
    i = pl.program_id(1)
    n = pts_ref.shape[1]
    pb = pts_ref[0, pl.ds(i * _M, _M), :]
    px = ptst_ref[0, 0:1, :]
    py = ptst_ref[0, 1:2, :]
    pz = ptst_ref[0, 2:3, :]
    pbx = pb[:, 0:1]
    pby = pb[:, 1:2]
    pbz = pb[:, 2:3]
    pb3 = pb[:, :3]

    p3 = pts_ref[0, :, :3]
    sq = (px * px + py * py + pz * pz)
    sqb = (pbx * pbx + pby * pby + pbz * pbz)
    dots = pbx * px + pby * py + pbz * pz
    d2 = (sqb + sq) - 2.0 * dots

    rows = jax.lax.broadcasted_iota(jnp.int32, (_M, n), 0) + i * _M
    cols = jax.lax.broadcasted_iota(jnp.int32, (_M, n), 1)
    inf = jnp.float32(jnp.inf)
    d2 = jnp.where(rows == cols, inf, d2)

    feats = []
    for _ in range(_K):
        m = jnp.min(d2, axis=1, keepdims=True)
        cand = jnp.where(d2 <= m, cols, n)
        fi = jnp.min(cand, axis=1, keepdims=True)
        oh = cols == fi
        d2 = jnp.where(oh, inf, d2)
        nbx = jnp.min(jnp.where(oh, px, inf), axis=1, keepdims=True)
        nby = jnp.min(jnp.where(oh, py, inf), axis=1, keepdims=True)
        nbz = jnp.min(jnp.where(oh, pz, inf), axis=1, keepdims=True)
        rx = nbx - pbx
        ry = nby - pby
        rz = nbz - pbz
        dist = jnp.sqrt(rx * rx + ry * ry + rz * rz + 1e-8)
        feats.append(jnp.concatenate([pb3, rx, ry, rz, dist], axis=1))

    f = jnp.concatenate(feats, axis=0)
    h = jnp.maximum(_mm(f, w1_ref[...]) + b1_ref[...], 0.0)
    h = jnp.maximum(_mm(h, w2_ref[...]) + b2_ref[...], 0.0)
    pooled = jnp.max(h.reshape(_K, _M, _OC), axis=0)

    lifted = jnp.maximum(_mm(pb3, w3_ref[...]) + b3_ref[...], 0.0)
    out = jnp.maximum(_mm(lifted, w4a_ref[...]) + _mm(pooled, w4b_ref[...])
                      + b4_ref[...], 0.0)
    out_ref[0] = out


def kernel(pts, W1, b1, W2, b2, W3, b3, W4, b4):
    b, n, _ = pts.shape
    pts8 = jnp.concatenate([pts, jnp.zeros((b, n, 5), pts.dtype)], axis=-1)
    ptst = jnp.concatenate(
        [jnp.swapaxes(pts, 1, 2), jnp.zeros((b, 5, n), pts.dtype)], axis=1)
    full = lambda bi, ii: (0, 0)
    out = pl.pallas_call(
        _nbr_kernel,
        grid=(b, n // _M),
        in_specs=[
            pl.BlockSpec((1, n, 8), lambda bi, ii: (bi, 0, 0)),
            pl.BlockSpec((1, 8, n), lambda bi, ii: (bi, 0, 0)),
            pl.BlockSpec((7, 16), full),
            pl.BlockSpec((1, 16), full),
            pl.BlockSpec((16, _OC), full),
            pl.BlockSpec((1, _OC), full),
            pl.BlockSpec((3, _OC), full),
            pl.BlockSpec((1, _OC), full),
            pl.BlockSpec((_OC, _OC), full),
            pl.BlockSpec((_OC, _OC), full),
            pl.BlockSpec((1, _OC), full),
        ],
        out_specs=pl.BlockSpec((1, _M, _OC), lambda bi, ii: (bi, ii, 0)),
        out_shape=jax.ShapeDtypeStruct((b, n, _OC), jnp.float32),
    )(pts8, ptst, W1, b1[None], W2, b2[None], W3, b3[None],
      W4[:_OC], W4[_OC:], b4[None])
    return out


# speedup vs baseline: 6.4889x; 1.0473x over previous
import jax
import jax.numpy as jnp
from jax.experimental import pallas as pl

_K = 16
_OC = 128
_M = 256


def _mm(a, b):
    return jax.lax.dot_general(a, b, (((1,), (0,)), ((), ())),
                               preferred_element_type=jnp.float32)


def _nbr_kernel(pts_ref, ptst_ref, w1_ref, b1_ref, w2_ref, b2_ref, w3_ref,
                b3_ref, w4a_ref, w4b_ref, b4_ref, out_ref):
    i = pl.program_id(1)
    n = pts_ref.shape[1]
    pb = pts_ref[0, pl.ds(i * _M, _M), :]
    px = ptst_ref[0, 0:1, :]
    py = ptst_ref[0, 1:2, :]
    pz = ptst_ref[0, 2:3, :]
    pbx = pb[:, 0:1]
    pby = pb[:, 1:2]
    pbz = pb[:, 2:3]
    pb3 = pb[:, :3]
    p3 = pts_ref[0, :, :3]

    sq = (px * px + py * py + pz * pz)
    sqb = (pbx * pbx + pby * pby + pbz * pbz)
    dots = jax.lax.dot_general(pb3.astype(jnp.bfloat16), p3.astype(jnp.bfloat16),
                               (((1,), (1,)), ((), ())),
                               preferred_element_type=jnp.float32)
    d2 = (sqb + sq) - 2.0 * dots

    cols = jax.lax.broadcasted_iota(jnp.int32, (_M, n), 1)
    inf = jnp.float32(jnp.inf)

    def _pop_min(d2):
        m = jnp.min(d2, axis=1, keepdims=True)
        cand = jnp.where(d2 <= m, cols, n)
        fi = jnp.min(cand, axis=1, keepdims=True)
        oh = cols == fi
        d2 = jnp.where(oh, inf, d2)
        nbx = jnp.min(jnp.where(oh, px, inf), axis=1, keepdims=True)
        nby = jnp.min(jnp.where(oh, py, inf), axis=1, keepdims=True)
        nbz = jnp.min(jnp.where(oh, pz, inf), axis=1, keepdims=True)
        return d2, nbx, nby, nbz

    d2, ax, ay, az = _pop_min(d2)
    a3 = jnp.concatenate([ax, ay, az], axis=1)

    feats = []
    for _ in range(_K):
        d2, nbx, nby, nbz = _pop_min(d2)
        rx = nbx - ax
        ry = nby - ay
        rz = nbz - az
        dist = jnp.sqrt(rx * rx + ry * ry + rz * rz + 1e-8)
        feats.append(jnp.concatenate([a3, rx, ry, rz, dist], axis=1))

    f = jnp.concatenate(feats, axis=0)
    h = jnp.maximum(_mm(f, w1_ref[...]) + b1_ref[...], 0.0)
    h = jnp.maximum(_mm(h, w2_ref[...]) + b2_ref[...], 0.0)
    pooled = jnp.max(h.reshape(_K, _M, _OC), axis=0)

    lifted = jnp.maximum(_mm(pb3, w3_ref[...]) + b3_ref[...], 0.0)
    out = jnp.maximum(_mm(lifted, w4a_ref[...]) + _mm(pooled, w4b_ref[...])
                      + b4_ref[...], 0.0)
    out_ref[0] = out


def kernel(pts, W1, b1, W2, b2, W3, b3, W4, b4):
    b, n, _ = pts.shape
    pts8 = jnp.concatenate([pts, jnp.zeros((b, n, 5), pts.dtype)], axis=-1)
    ptst = jnp.concatenate(
        [jnp.swapaxes(pts, 1, 2), jnp.zeros((b, 5, n), pts.dtype)], axis=1)
    full = lambda bi, ii: (0, 0)
    out = pl.pallas_call(
        _nbr_kernel,
        grid=(b, n // _M),
        in_specs=[
            pl.BlockSpec((1, n, 8), lambda bi, ii: (bi, 0, 0)),
            pl.BlockSpec((1, 8, n), lambda bi, ii: (bi, 0, 0)),
            pl.BlockSpec((7, 16), full),
            pl.BlockSpec((1, 16), full),
            pl.BlockSpec((16, _OC), full),
            pl.BlockSpec((1, _OC), full),
            pl.BlockSpec((3, _OC), full),
            pl.BlockSpec((1, _OC), full),
            pl.BlockSpec((_OC, _OC), full),
            pl.BlockSpec((_OC, _OC), full),
            pl.BlockSpec((1, _OC), full),
        ],
        out_specs=pl.BlockSpec((1, _M, _OC), lambda bi, ii: (bi, ii, 0)),
        out_shape=jax.ShapeDtypeStruct((b, n, _OC), jnp.float32),
    )(pts8, ptst, W1, b1[None], W2, b2[None], W3, b3[None],
      W4[:_OC], W4[_OC:], b4[None])
    return out


# speedup vs baseline: 7.2151x; 1.1119x over previous
import jax
import jax.numpy as jnp
from jax.experimental import pallas as pl

_K = 16
_OC = 128
_M = 512


def _mm(a, b):
    return jax.lax.dot_general(a, b, (((1,), (0,)), ((), ())),
                               preferred_element_type=jnp.float32)


def _nbr_kernel(pts_ref, ptst_ref, w1_ref, b1_ref, w2_ref, b2_ref, w3_ref,
                b3_ref, w4a_ref, w4b_ref, b4_ref, out_ref):
    i = pl.program_id(1)
    n = pts_ref.shape[1]
    pb = pts_ref[0, pl.ds(i * _M, _M), :]
    px = ptst_ref[0, 0:1, :]
    py = ptst_ref[0, 1:2, :]
    pz = ptst_ref[0, 2:3, :]
    pbx = pb[:, 0:1]
    pby = pb[:, 1:2]
    pbz = pb[:, 2:3]
    pb3 = pb[:, :3]
    p3 = pts_ref[0, :, :3]

    sq = (px * px + py * py + pz * pz)
    sqb = (pbx * pbx + pby * pby + pbz * pbz)
    dots = jax.lax.dot_general(pb3.astype(jnp.bfloat16), p3.astype(jnp.bfloat16),
                               (((1,), (1,)), ((), ())),
                               preferred_element_type=jnp.float32)
    d2 = (sqb + sq) - 2.0 * dots

    cols = jax.lax.broadcasted_iota(jnp.int32, (_M, n), 1)
    inf = jnp.float32(jnp.inf)

    def _pop_min(d2):
        m = jnp.min(d2, axis=1, keepdims=True)
        cand = jnp.where(d2 <= m, cols, n)
        fi = jnp.min(cand, axis=1, keepdims=True)
        oh = cols == fi
        d2 = jnp.where(oh, inf, d2)
        nbx = jnp.min(jnp.where(oh, px, inf), axis=1, keepdims=True)
        nby = jnp.min(jnp.where(oh, py, inf), axis=1, keepdims=True)
        nbz = jnp.min(jnp.where(oh, pz, inf), axis=1, keepdims=True)
        return d2, nbx, nby, nbz

    d2, ax, ay, az = _pop_min(d2)
    a3 = jnp.concatenate([ax, ay, az], axis=1)

    feats = []
    for _ in range(_K):
        d2, nbx, nby, nbz = _pop_min(d2)
        rx = nbx - ax
        ry = nby - ay
        rz = nbz - az
        dist = jnp.sqrt(rx * rx + ry * ry + rz * rz + 1e-8)
        feats.append(jnp.concatenate([a3, rx, ry, rz, dist], axis=1))

    f = jnp.concatenate(feats, axis=0)
    h = jnp.maximum(_mm(f, w1_ref[...]) + b1_ref[...], 0.0)
    h = jnp.maximum(_mm(h, w2_ref[...]) + b2_ref[...], 0.0)
    pooled = jnp.max(h.reshape(_K, _M, _OC), axis=0)

    lifted = jnp.maximum(_mm(pb3, w3_ref[...]) + b3_ref[...], 0.0)
    out = jnp.maximum(_mm(lifted, w4a_ref[...]) + _mm(pooled, w4b_ref[...])
                      + b4_ref[...], 0.0)
    out_ref[0] = out


def kernel(pts, W1, b1, W2, b2, W3, b3, W4, b4):
    b, n, _ = pts.shape
    pts8 = jnp.concatenate([pts, jnp.zeros((b, n, 5), pts.dtype)], axis=-1)
    ptst = jnp.concatenate(
        [jnp.swapaxes(pts, 1, 2), jnp.zeros((b, 5, n), pts.dtype)], axis=1)
    full = lambda bi, ii: (0, 0)
    out = pl.pallas_call(
        _nbr_kernel,
        grid=(b, n // _M),
        in_specs=[
            pl.BlockSpec((1, n, 8), lambda bi, ii: (bi, 0, 0)),
            pl.BlockSpec((1, 8, n), lambda bi, ii: (bi, 0, 0)),
            pl.BlockSpec((7, 16), full),
            pl.BlockSpec((1, 16), full),
            pl.BlockSpec((16, _OC), full),
            pl.BlockSpec((1, _OC), full),
            pl.BlockSpec((3, _OC), full),
            pl.BlockSpec((1, _OC), full),
            pl.BlockSpec((_OC, _OC), full),
            pl.BlockSpec((_OC, _OC), full),
            pl.BlockSpec((1, _OC), full),
        ],
        out_specs=pl.BlockSpec((1, _M, _OC), lambda bi, ii: (bi, ii, 0)),
        out_shape=jax.ShapeDtypeStruct((b, n, _OC), jnp.float32),
    )(pts8, ptst, W1, b1[None], W2, b2[None], W3, b3[None],
      W4[:_OC], W4[_OC:], b4[None])
    return out


# speedup vs baseline: 7.5008x; 1.0396x over previous
import jax
import jax.numpy as jnp
from jax.experimental import pallas as pl

_K = 16
_OC = 128
_M = 1024


def _mm(a, b):
    return jax.lax.dot_general(a, b, (((1,), (0,)), ((), ())),
                               preferred_element_type=jnp.float32)


def _nbr_kernel(pts_ref, ptst_ref, w1_ref, b1_ref, w2_ref, b2_ref, w3_ref,
                b3_ref, w4a_ref, w4b_ref, b4_ref, out_ref):
    i = pl.program_id(1)
    n = pts_ref.shape[1]
    pb = pts_ref[0, pl.ds(i * _M, _M), :]
    px = ptst_ref[0, 0:1, :]
    py = ptst_ref[0, 1:2, :]
    pz = ptst_ref[0, 2:3, :]
    pbx = pb[:, 0:1]
    pby = pb[:, 1:2]
    pbz = pb[:, 2:3]
    pb3 = pb[:, :3]
    p3 = pts_ref[0, :, :3]

    sq = (px * px + py * py + pz * pz)
    sqb = (pbx * pbx + pby * pby + pbz * pbz)
    dots = jax.lax.dot_general(pb3.astype(jnp.bfloat16), p3.astype(jnp.bfloat16),
                               (((1,), (1,)), ((), ())),
                               preferred_element_type=jnp.float32)
    d2 = (sqb + sq) - 2.0 * dots

    cols = jax.lax.broadcasted_iota(jnp.int32, (_M, n), 1)
    inf = jnp.float32(jnp.inf)

    def _pop_min(d2):
        m = jnp.min(d2, axis=1, keepdims=True)
        cand = jnp.where(d2 <= m, cols, n)
        fi = jnp.min(cand, axis=1, keepdims=True)
        oh = cols == fi
        d2 = jnp.where(oh, inf, d2)
        nbx = jnp.min(jnp.where(oh, px, inf), axis=1, keepdims=True)
        nby = jnp.min(jnp.where(oh, py, inf), axis=1, keepdims=True)
        nbz = jnp.min(jnp.where(oh, pz, inf), axis=1, keepdims=True)
        return d2, nbx, nby, nbz

    d2, ax, ay, az = _pop_min(d2)
    a3 = jnp.concatenate([ax, ay, az], axis=1)

    feats = []
    for _ in range(_K):
        d2, nbx, nby, nbz = _pop_min(d2)
        rx = nbx - ax
        ry = nby - ay
        rz = nbz - az
        dist = jnp.sqrt(rx * rx + ry * ry + rz * rz + 1e-8)
        feats.append(jnp.concatenate([a3, rx, ry, rz, dist], axis=1))

    f = jnp.concatenate(feats, axis=0)
    h = jnp.maximum(_mm(f, w1_ref[...]) + b1_ref[...], 0.0)
    h = jnp.maximum(_mm(h, w2_ref[...]) + b2_ref[...], 0.0)
    pooled = jnp.max(h.reshape(_K, _M, _OC), axis=0)

    lifted = jnp.maximum(_mm(pb3, w3_ref[...]) + b3_ref[...], 0.0)
    out = jnp.maximum(_mm(lifted, w4a_ref[...]) + _mm(pooled, w4b_ref[...])
                      + b4_ref[...], 0.0)
    out_ref[0] = out


def kernel(pts, W1, b1, W2, b2, W3, b3, W4, b4):
    b, n, _ = pts.shape
    pts8 = jnp.concatenate([pts, jnp.zeros((b, n, 5), pts.dtype)], axis=-1)
    ptst = jnp.concatenate(
        [jnp.swapaxes(pts, 1, 2), jnp.zeros((b, 5, n), pts.dtype)], axis=1)
    full = lambda bi, ii: (0, 0)
    out = pl.pallas_call(
        _nbr_kernel,
        grid=(b, n // _M),
        in_specs=[
            pl.BlockSpec((1, n, 8), lambda bi, ii: (bi, 0, 0)),
            pl.BlockSpec((1, 8, n), lambda bi, ii: (bi, 0, 0)),
            pl.BlockSpec((7, 16), full),
            pl.BlockSpec((1, 16), full),
            pl.BlockSpec((16, _OC), full),
            pl.BlockSpec((1, _OC), full),
            pl.BlockSpec((3, _OC), full),
            pl.BlockSpec((1, _OC), full),
            pl.BlockSpec((_OC, _OC), full),
            pl.BlockSpec((_OC, _OC), full),
            pl.BlockSpec((1, _OC), full),
        ],
        out_specs=pl.BlockSpec((1, _M, _OC), lambda bi, ii: (bi, ii, 0)),
        out_shape=jax.ShapeDtypeStruct((b, n, _OC), jnp.float32),
    )(pts8, ptst, W1, b1[None], W2, b2[None], W3, b3[None],
      W4[:_OC], W4[_OC:], b4[None])
    return out
